# Initial kernel scaffold; baseline (speedup 1.0000x reference)
import jax
import jax.numpy as jnp
from jax.experimental import pallas as pl


def kernel(h2d, h3d, mask, W_in, b_in, W_gate, b_gate, W1, b1, W2, b2, W_out, b_out):
    raise NotImplementedError("write your pallas kernel here")


# speedup vs baseline: 1.8284x; 1.8284x over previous
import jax
import jax.numpy as jnp
from jax import lax
from jax.experimental import pallas as pl
from jax.experimental.pallas import tpu as pltpu
from jax.experimental.pallas import tpu_sc as plsc

B, A = 4, 2048
N = B * A
D2D, D3D, DF, DFF = 768, 768, 1024, 2048
E, TEMP, OUT = 8, 1.2, 768

BLK = 256
NP = 2 * N + E * BLK
NB = NP // BLK

NW = 32
TPW = N // NW
SUB = 32
NSUB = TPW // SUB
KC = 2 * NSUB

BT = 1024


def _k1_body(h2_ref, h3_ref, Win_ref, bin_ref, Wg_ref, bg_ref,
             x_ref, oh_ref, wp_ref):
    x = (jnp.dot(h2_ref[...], Win_ref[:D2D], preferred_element_type=jnp.float32)
         + jnp.dot(h3_ref[...], Win_ref[D2D:], preferred_element_type=jnp.float32)
         + bin_ref[...])
    x_ref[...] = x
    l = (jnp.dot(x, Wg_ref[...], preferred_element_type=jnp.float32)
         + bg_ref[...]) * (1.0 / TEMP)
    iota = lax.broadcasted_iota(jnp.int32, (BT, E), 1)
    m1 = jnp.max(l, axis=1, keepdims=True)
    i1 = jnp.min(jnp.where(l == m1, iota, E), axis=1, keepdims=True)
    lm = jnp.where(iota == i1, -1e30, l)
    m2 = jnp.max(lm, axis=1, keepdims=True)
    i2 = jnp.min(jnp.where(lm == m2, iota, E), axis=1, keepdims=True)
    ex = jnp.exp(l - m1)
    z = jnp.sum(ex, axis=1, keepdims=True)
    p1 = jnp.max(ex, axis=1, keepdims=True) / z
    p2 = jnp.sum(jnp.where(iota == i2, ex, 0.0), axis=1, keepdims=True) / z
    s = p1 + p2 + 1e-9
    w1 = p1 / s
    w2 = p2 / s
    lo_first = i1 < i2
    wp_ref[...] = jnp.concatenate(
        [jnp.where(lo_first, w1, w2), jnp.where(lo_first, w2, w1)], axis=1)
    oh_ref[...] = ((iota == jnp.minimum(i1, i2))
                   | (iota == jnp.maximum(i1, i2))).astype(jnp.float32)


def _k2_body(oh_ref, d2_ref, cnt_ref):
    oh = oh_ref[...]
    inc = oh
    off = 1
    while off < N:
        inc = inc + jnp.concatenate(
            [jnp.zeros((off, E), jnp.float32), inc[:-off]], axis=0)
        off *= 2
    excl = inc - oh
    counts = inc[N - 1:N, :]
    pc = jnp.ceil(counts * (1.0 / BLK)) * BLK
    po = jnp.concatenate([jnp.zeros((1, 1), jnp.float32), pc[:, :-1]], axis=1)
    off = 1
    while off < E:
        po = po + jnp.concatenate(
            [jnp.zeros((1, off), jnp.float32), po[:, :-off]], axis=1)
        off *= 2
    dest = po + excl
    active = oh > 0.0
    dlo = jnp.min(jnp.where(active, dest, 1e9), axis=1, keepdims=True)
    dhi = jnp.max(jnp.where(active, dest, -1.0), axis=1, keepdims=True)
    d2_ref[...] = jnp.concatenate([dlo, dhi], axis=1).astype(jnp.int32)
    cnt_ref[...] = counts.astype(jnp.int32)


def _sc_scatter_body(x_hbm, didx_hbm, xg_hbm, didx_v, xbuf, sem):
    w = lax.axis_index("s") * 2 + lax.axis_index("c")
    pltpu.sync_copy(didx_hbm.at[w], didx_v)
    for sub in range(NSUB):
        base = w * TPW + sub * SUB
        pltpu.sync_copy(x_hbm.at[pl.ds(base, SUB)], xbuf)
        pltpu.async_copy(xbuf, xg_hbm.at[didx_v.at[2 * sub]], sem).wait()
        pltpu.async_copy(xbuf, xg_hbm.at[didx_v.at[2 * sub + 1]], sem).wait()


def _sc_gather_body(yg_hbm, didx_hbm, c0_hbm, c1_hbm, didx_v, buf, sem):
    w = lax.axis_index("s") * 2 + lax.axis_index("c")
    pltpu.sync_copy(didx_hbm.at[w], didx_v)
    for sub in range(NSUB):
        base = w * TPW + sub * SUB
        pltpu.async_copy(yg_hbm.at[didx_v.at[2 * sub]], buf, sem).wait()
        pltpu.sync_copy(buf, c0_hbm.at[pl.ds(base, SUB)])
        pltpu.async_copy(yg_hbm.at[didx_v.at[2 * sub + 1]], buf, sem).wait()
        pltpu.sync_copy(buf, c1_hbm.at[pl.ds(base, SUB)])


def _sc_mesh():
    return plsc.VectorSubcoreMesh(core_axis_name="c", subcore_axis_name="s")


def _scatter_call(x, didx):
    return pl.kernel(
        _sc_scatter_body,
        out_type=jax.ShapeDtypeStruct((NP, DF), jnp.float32),
        mesh=_sc_mesh(),
        scratch_types=[
            pltpu.VMEM((KC, SUB), jnp.int32),
            pltpu.VMEM((SUB, DF), jnp.float32),
            pltpu.SemaphoreType.DMA,
        ],
    )(x, didx)


def _gather_call(yg, didx):
    return pl.kernel(
        _sc_gather_body,
        out_type=(
            jax.ShapeDtypeStruct((N, DF), jnp.float32),
            jax.ShapeDtypeStruct((N, DF), jnp.float32),
        ),
        mesh=_sc_mesh(),
        scratch_types=[
            pltpu.VMEM((KC, SUB), jnp.int32),
            pltpu.VMEM((SUB, DF), jnp.float32),
            pltpu.SemaphoreType.DMA,
        ],
    )(yg, didx)


def _k4_body(be_ref, xg_ref, W1_ref, b1_ref, W2_ref, b2_ref, yg_ref):
    x = xg_ref[...]
    u = jnp.dot(x, W1_ref[0], preferred_element_type=jnp.float32) + b1_ref[0]
    h = 0.5 * u * (1.0 + jnp.tanh(0.7978845608028654 * (u + 0.044715 * u * u * u)))
    yg_ref[...] = (jnp.dot(h, W2_ref[0], preferred_element_type=jnp.float32)
                   + b2_ref[0])


def _k6_body(c0_ref, c1_ref, wp_ref, mk_ref, Wo_ref, bo_ref, y_ref):
    wp = wp_ref[...]
    comb = (wp[:, :1] * c0_ref[...] + wp[:, 1:2] * c1_ref[...]) * mk_ref[...]
    y_ref[...] = (jnp.dot(comb, Wo_ref[...], preferred_element_type=jnp.float32)
                  + bo_ref[...])


def kernel(h2d, h3d, mask, W_in, b_in, W_gate, b_gate, W1, b1, W2, b2, W_out, b_out):
    h2 = h2d.reshape(N, D2D)
    h3 = h3d.reshape(N, D3D)
    maskf = mask.reshape(N, 1).astype(jnp.float32)

    x, oh, wp = pl.pallas_call(
        _k1_body,
        grid=(N // BT,),
        in_specs=[
            pl.BlockSpec((BT, D2D), lambda i: (i, 0)),
            pl.BlockSpec((BT, D3D), lambda i: (i, 0)),
            pl.BlockSpec((D2D + D3D, DF), lambda i: (0, 0)),
            pl.BlockSpec((1, DF), lambda i: (0, 0)),
            pl.BlockSpec((DF, E), lambda i: (0, 0)),
            pl.BlockSpec((1, E), lambda i: (0, 0)),
        ],
        out_specs=[
            pl.BlockSpec((BT, DF), lambda i: (i, 0)),
            pl.BlockSpec((BT, E), lambda i: (i, 0)),
            pl.BlockSpec((BT, 2), lambda i: (i, 0)),
        ],
        out_shape=[
            jax.ShapeDtypeStruct((N, DF), jnp.float32),
            jax.ShapeDtypeStruct((N, E), jnp.float32),
            jax.ShapeDtypeStruct((N, 2), jnp.float32),
        ],
    )(h2, h3, W_in, b_in.reshape(1, DF), W_gate, b_gate.reshape(1, E))

    d2, counts = pl.pallas_call(
        _k2_body,
        out_shape=[
            jax.ShapeDtypeStruct((N, 2), jnp.int32),
            jax.ShapeDtypeStruct((1, E), jnp.int32),
        ],
    )(oh)

    cnt = counts.reshape(E)
    pci = ((cnt + BLK - 1) // BLK) * BLK
    ends = jnp.cumsum(pci)
    bidx = jnp.arange(NB, dtype=jnp.int32) * BLK
    be = jnp.minimum(
        jnp.sum((bidx[:, None] >= ends[None, :]).astype(jnp.int32), axis=1),
        E - 1).astype(jnp.int32)

    didx = d2.reshape(NW, NSUB, SUB, 2).transpose(0, 1, 3, 2).reshape(NW, KC, SUB)

    xg = _scatter_call(x, didx)

    yg = pl.pallas_call(
        _k4_body,
        grid_spec=pltpu.PrefetchScalarGridSpec(
            num_scalar_prefetch=1,
            grid=(NB,),
            in_specs=[
                pl.BlockSpec((BLK, DF), lambda b, be_r: (b, 0)),
                pl.BlockSpec((1, DF, DFF), lambda b, be_r: (be_r[b], 0, 0)),
                pl.BlockSpec((1, 1, DFF), lambda b, be_r: (be_r[b], 0, 0)),
                pl.BlockSpec((1, DFF, DF), lambda b, be_r: (be_r[b], 0, 0)),
                pl.BlockSpec((1, 1, DF), lambda b, be_r: (be_r[b], 0, 0)),
            ],
            out_specs=pl.BlockSpec((BLK, DF), lambda b, be_r: (b, 0)),
        ),
        out_shape=jax.ShapeDtypeStruct((NP, DF), jnp.float32),
    )(be, xg, W1, b1.reshape(E, 1, DFF), W2, b2.reshape(E, 1, DF))

    c0, c1 = _gather_call(yg, didx)

    y = pl.pallas_call(
        _k6_body,
        grid=(N // BT,),
        in_specs=[
            pl.BlockSpec((BT, DF), lambda i: (i, 0)),
            pl.BlockSpec((BT, DF), lambda i: (i, 0)),
            pl.BlockSpec((BT, 2), lambda i: (i, 0)),
            pl.BlockSpec((BT, 1), lambda i: (i, 0)),
            pl.BlockSpec((DF, OUT), lambda i: (0, 0)),
            pl.BlockSpec((1, OUT), lambda i: (0, 0)),
        ],
        out_specs=pl.BlockSpec((BT, OUT), lambda i: (i, 0)),
        out_shape=jax.ShapeDtypeStruct((N, OUT), jnp.float32),
    )(c0, c1, wp, maskf, W_out, b_out.reshape(1, OUT))

    return y.reshape(B, A, OUT)


# speedup vs baseline: 1.8511x; 1.0125x over previous
import jax
import jax.numpy as jnp
from jax import lax
from jax.experimental import pallas as pl
from jax.experimental.pallas import tpu as pltpu
from jax.experimental.pallas import tpu_sc as plsc

B, A = 4, 2048
N = B * A
D2D, D3D, DF, DFF = 768, 768, 1024, 2048
E, TEMP, OUT = 8, 1.2, 768

BLK = 256
NP = 2 * N + E * BLK
NB = NP // BLK

NW = 32
TPW = N // NW
SUB = 32
NSUB = TPW // SUB
KC = 2 * NSUB

BT = 1024


def _k1_body(h2_ref, h3_ref, Win_ref, bin_ref, Wg_ref, bg_ref,
             x_ref, oh_ref, wp_ref):
    x = (jnp.dot(h2_ref[...], Win_ref[:D2D], preferred_element_type=jnp.float32)
         + jnp.dot(h3_ref[...], Win_ref[D2D:], preferred_element_type=jnp.float32)
         + bin_ref[...])
    x_ref[...] = x
    l = (jnp.dot(x, Wg_ref[...], preferred_element_type=jnp.float32)
         + bg_ref[...]) * (1.0 / TEMP)
    iota = lax.broadcasted_iota(jnp.int32, (BT, E), 1)
    m1 = jnp.max(l, axis=1, keepdims=True)
    i1 = jnp.min(jnp.where(l == m1, iota, E), axis=1, keepdims=True)
    lm = jnp.where(iota == i1, -1e30, l)
    m2 = jnp.max(lm, axis=1, keepdims=True)
    i2 = jnp.min(jnp.where(lm == m2, iota, E), axis=1, keepdims=True)
    ex = jnp.exp(l - m1)
    z = jnp.sum(ex, axis=1, keepdims=True)
    p1 = jnp.max(ex, axis=1, keepdims=True) / z
    p2 = jnp.sum(jnp.where(iota == i2, ex, 0.0), axis=1, keepdims=True) / z
    s = p1 + p2 + 1e-9
    w1 = p1 / s
    w2 = p2 / s
    lo_first = i1 < i2
    wp_ref[...] = jnp.concatenate(
        [jnp.where(lo_first, w1, w2), jnp.where(lo_first, w2, w1)], axis=1)
    oh_ref[...] = ((iota == jnp.minimum(i1, i2))
                   | (iota == jnp.maximum(i1, i2))).astype(jnp.float32)


def _k2_body(oh_ref, dlo_ref, dhi_ref, be_ref):
    oh = oh_ref[...]
    inc = oh
    off = 1
    while off < N:
        inc = inc + jnp.concatenate(
            [jnp.zeros((off, E), jnp.float32), inc[:-off]], axis=0)
        off *= 2
    excl = inc - oh
    counts = inc[N - 1:N, :]
    pc = jnp.ceil(counts * (1.0 / BLK)) * BLK
    ends = pc
    off = 1
    while off < E:
        ends = ends + jnp.concatenate(
            [jnp.zeros((1, off), jnp.float32), ends[:, :-off]], axis=1)
        off *= 2
    dest = (ends - pc) + excl
    active = oh > 0.0
    dlo_ref[...] = jnp.min(
        jnp.where(active, dest, 1e9), axis=1, keepdims=True).astype(jnp.int32)
    dhi_ref[...] = jnp.max(
        jnp.where(active, dest, -1.0), axis=1, keepdims=True).astype(jnp.int32)
    bio = lax.broadcasted_iota(jnp.int32, (1, 128), 1).astype(jnp.float32) * BLK
    acc = jnp.zeros((1, 128), jnp.float32)
    for e in range(E):
        acc = acc + (bio >= ends[:, e:e + 1]).astype(jnp.float32)
    be_ref[...] = jnp.minimum(acc, float(E - 1)).astype(jnp.int32)


def _sc_scatter_body(x_hbm, dlo_hbm, dhi_hbm, xg_hbm,
                     dlo_v, dhi_v, buf0, buf1, lsem0, lsem1, ssem0, ssem1):
    w = lax.axis_index("s") * 2 + lax.axis_index("c")
    base = w * TPW
    pltpu.sync_copy(dlo_hbm.at[w], dlo_v)
    pltpu.sync_copy(dhi_hbm.at[w], dhi_v)
    bufs = (buf0, buf1)
    lsems = (lsem0, lsem1)
    ssems = (ssem0, ssem1)
    loads = [None] * NSUB
    scat = []
    loads[0] = pltpu.async_copy(x_hbm.at[pl.ds(base, SUB)], bufs[0], lsems[0])
    for sub in range(NSUB):
        cur = sub & 1
        nxt = 1 - cur
        loads[sub].wait()
        if sub + 1 < NSUB:
            if sub >= 1:
                scat[2 * sub - 2].wait()
                scat[2 * sub - 1].wait()
            loads[sub + 1] = pltpu.async_copy(
                x_hbm.at[pl.ds(base + (sub + 1) * SUB, SUB)], bufs[nxt], lsems[nxt])
        scat.append(pltpu.async_copy(bufs[cur], xg_hbm.at[dlo_v.at[sub]], ssems[cur]))
        scat.append(pltpu.async_copy(bufs[cur], xg_hbm.at[dhi_v.at[sub]], ssems[cur]))
    for t in range(4):
        scat[2 * NSUB - 4 + t].wait()


def _sc_gather_body(yg_hbm, dlo_hbm, dhi_hbm, c0_hbm, c1_hbm,
                    dlo_v, dhi_v, buf0, buf1, gsem0, gsem1, wsem0, wsem1):
    w = lax.axis_index("s") * 2 + lax.axis_index("c")
    base = w * TPW
    pltpu.sync_copy(dlo_hbm.at[w], dlo_v)
    pltpu.sync_copy(dhi_hbm.at[w], dhi_v)
    bufs = (buf0, buf1)
    gsems = (gsem0, gsem1)
    wsems = (wsem0, wsem1)
    NT = 2 * NSUB

    def idx_ref(t):
        return (dlo_v if t % 2 == 0 else dhi_v).at[t // 2]

    def out_ref(t):
        dst = c0_hbm if t % 2 == 0 else c1_hbm
        return dst.at[pl.ds(base + (t // 2) * SUB, SUB)]

    g = [None] * NT
    wrt = [None] * NT
    g[0] = pltpu.async_copy(yg_hbm.at[idx_ref(0)], bufs[0], gsems[0])
    for t in range(NT):
        cur = t & 1
        nxt = 1 - cur
        g[t].wait()
        if t + 1 < NT:
            if t >= 1:
                wrt[t - 1].wait()
            g[t + 1] = pltpu.async_copy(yg_hbm.at[idx_ref(t + 1)], bufs[nxt], gsems[nxt])
        wrt[t] = pltpu.async_copy(bufs[cur], out_ref(t), wsems[cur])
    wrt[NT - 2].wait()
    wrt[NT - 1].wait()


def _sc_mesh():
    return plsc.VectorSubcoreMesh(core_axis_name="c", subcore_axis_name="s")


def _scatter_call(x, dlo3, dhi3):
    return pl.kernel(
        _sc_scatter_body,
        out_type=jax.ShapeDtypeStruct((NP, DF), jnp.float32),
        mesh=_sc_mesh(),
        scratch_types=[
            pltpu.VMEM((NSUB, SUB), jnp.int32),
            pltpu.VMEM((NSUB, SUB), jnp.int32),
            pltpu.VMEM((SUB, DF), jnp.float32),
            pltpu.VMEM((SUB, DF), jnp.float32),
            pltpu.SemaphoreType.DMA,
            pltpu.SemaphoreType.DMA,
            pltpu.SemaphoreType.DMA,
            pltpu.SemaphoreType.DMA,
        ],
    )(x, dlo3, dhi3)


def _gather_call(yg, dlo3, dhi3):
    return pl.kernel(
        _sc_gather_body,
        out_type=(
            jax.ShapeDtypeStruct((N, DF), jnp.float32),
            jax.ShapeDtypeStruct((N, DF), jnp.float32),
        ),
        mesh=_sc_mesh(),
        scratch_types=[
            pltpu.VMEM((NSUB, SUB), jnp.int32),
            pltpu.VMEM((NSUB, SUB), jnp.int32),
            pltpu.VMEM((SUB, DF), jnp.float32),
            pltpu.VMEM((SUB, DF), jnp.float32),
            pltpu.SemaphoreType.DMA,
            pltpu.SemaphoreType.DMA,
            pltpu.SemaphoreType.DMA,
            pltpu.SemaphoreType.DMA,
        ],
    )(yg, dlo3, dhi3)


def _k4_body(be_ref, xg_ref, W1_ref, b1_ref, W2_ref, b2_ref, yg_ref):
    x = xg_ref[...]
    u = jnp.dot(x, W1_ref[0], preferred_element_type=jnp.float32) + b1_ref[0]
    h = 0.5 * u * (1.0 + jnp.tanh(0.7978845608028654 * (u + 0.044715 * u * u * u)))
    yg_ref[...] = (jnp.dot(h, W2_ref[0], preferred_element_type=jnp.float32)
                   + b2_ref[0])


def _k6_body(c0_ref, c1_ref, wp_ref, mk_ref, Wo_ref, bo_ref, y_ref):
    wp = wp_ref[...]
    comb = (wp[:, :1] * c0_ref[...] + wp[:, 1:2] * c1_ref[...]) * mk_ref[...]
    y_ref[...] = (jnp.dot(comb, Wo_ref[...], preferred_element_type=jnp.float32)
                  + bo_ref[...])


def kernel(h2d, h3d, mask, W_in, b_in, W_gate, b_gate, W1, b1, W2, b2, W_out, b_out):
    h2 = h2d.reshape(N, D2D)
    h3 = h3d.reshape(N, D3D)
    maskf = mask.reshape(N, 1).astype(jnp.float32)

    x, oh, wp = pl.pallas_call(
        _k1_body,
        grid=(N // BT,),
        in_specs=[
            pl.BlockSpec((BT, D2D), lambda i: (i, 0)),
            pl.BlockSpec((BT, D3D), lambda i: (i, 0)),
            pl.BlockSpec((D2D + D3D, DF), lambda i: (0, 0)),
            pl.BlockSpec((1, DF), lambda i: (0, 0)),
            pl.BlockSpec((DF, E), lambda i: (0, 0)),
            pl.BlockSpec((1, E), lambda i: (0, 0)),
        ],
        out_specs=[
            pl.BlockSpec((BT, DF), lambda i: (i, 0)),
            pl.BlockSpec((BT, E), lambda i: (i, 0)),
            pl.BlockSpec((BT, 2), lambda i: (i, 0)),
        ],
        out_shape=[
            jax.ShapeDtypeStruct((N, DF), jnp.float32),
            jax.ShapeDtypeStruct((N, E), jnp.float32),
            jax.ShapeDtypeStruct((N, 2), jnp.float32),
        ],
    )(h2, h3, W_in, b_in.reshape(1, DF), W_gate, b_gate.reshape(1, E))

    dlo, dhi, be2 = pl.pallas_call(
        _k2_body,
        out_shape=[
            jax.ShapeDtypeStruct((N, 1), jnp.int32),
            jax.ShapeDtypeStruct((N, 1), jnp.int32),
            jax.ShapeDtypeStruct((1, 128), jnp.int32),
        ],
    )(oh)

    be = be2.reshape(128)
    dlo3 = dlo.reshape(NW, NSUB, SUB)
    dhi3 = dhi.reshape(NW, NSUB, SUB)

    xg = _scatter_call(x, dlo3, dhi3)

    yg = pl.pallas_call(
        _k4_body,
        grid_spec=pltpu.PrefetchScalarGridSpec(
            num_scalar_prefetch=1,
            grid=(NB,),
            in_specs=[
                pl.BlockSpec((BLK, DF), lambda b, be_r: (b, 0)),
                pl.BlockSpec((1, DF, DFF), lambda b, be_r: (be_r[b], 0, 0)),
                pl.BlockSpec((1, 1, DFF), lambda b, be_r: (be_r[b], 0, 0)),
                pl.BlockSpec((1, DFF, DF), lambda b, be_r: (be_r[b], 0, 0)),
                pl.BlockSpec((1, 1, DF), lambda b, be_r: (be_r[b], 0, 0)),
            ],
            out_specs=pl.BlockSpec((BLK, DF), lambda b, be_r: (b, 0)),
        ),
        out_shape=jax.ShapeDtypeStruct((NP, DF), jnp.float32),
    )(be, xg, W1, b1.reshape(E, 1, DFF), W2, b2.reshape(E, 1, DF))

    c0, c1 = _gather_call(yg, dlo3, dhi3)

    y = pl.pallas_call(
        _k6_body,
        grid=(N // BT,),
        in_specs=[
            pl.BlockSpec((BT, DF), lambda i: (i, 0)),
            pl.BlockSpec((BT, DF), lambda i: (i, 0)),
            pl.BlockSpec((BT, 2), lambda i: (i, 0)),
            pl.BlockSpec((BT, 1), lambda i: (i, 0)),
            pl.BlockSpec((DF, OUT), lambda i: (0, 0)),
            pl.BlockSpec((1, OUT), lambda i: (0, 0)),
        ],
        out_specs=pl.BlockSpec((BT, OUT), lambda i: (i, 0)),
        out_shape=jax.ShapeDtypeStruct((N, OUT), jnp.float32),
    )(c0, c1, wp, maskf, W_out, b_out.reshape(1, OUT))

    return y.reshape(B, A, OUT)


# speedup vs baseline: 1.9244x; 1.0396x over previous
import jax
import jax.numpy as jnp
from jax import lax
from jax.experimental import pallas as pl
from jax.experimental.pallas import tpu as pltpu
from jax.experimental.pallas import tpu_sc as plsc

B, A = 4, 2048
N = B * A
D2D, D3D, DF, DFF = 768, 768, 1024, 2048
E, TEMP, OUT = 8, 1.2, 768

BLK = 256
NP = 2 * N + E * BLK
NB = NP // BLK

NW = 32
TPW = N // NW
SUB = 32
NSUB = TPW // SUB
KC = 2 * NSUB

BT = 1024


def _k1_body(h2_ref, h3_ref, Win_ref, bin_ref, Wg_ref, bg_ref,
             x_ref, oh_ref, wp_ref):
    x = (jnp.dot(h2_ref[...], Win_ref[:D2D], preferred_element_type=jnp.float32)
         + jnp.dot(h3_ref[...], Win_ref[D2D:], preferred_element_type=jnp.float32)
         + bin_ref[...])
    x_ref[...] = x
    l = (jnp.dot(x, Wg_ref[...], preferred_element_type=jnp.float32)
         + bg_ref[...]) * (1.0 / TEMP)
    iota = lax.broadcasted_iota(jnp.int32, (BT, E), 1)
    m1 = jnp.max(l, axis=1, keepdims=True)
    i1 = jnp.min(jnp.where(l == m1, iota, E), axis=1, keepdims=True)
    lm = jnp.where(iota == i1, -1e30, l)
    m2 = jnp.max(lm, axis=1, keepdims=True)
    i2 = jnp.min(jnp.where(lm == m2, iota, E), axis=1, keepdims=True)
    ex = jnp.exp(l - m1)
    z = jnp.sum(ex, axis=1, keepdims=True)
    p1 = jnp.max(ex, axis=1, keepdims=True) / z
    p2 = jnp.sum(jnp.where(iota == i2, ex, 0.0), axis=1, keepdims=True) / z
    s = p1 + p2 + 1e-9
    w1 = p1 / s
    w2 = p2 / s
    lo_first = i1 < i2
    wp_ref[...] = jnp.concatenate(
        [jnp.where(lo_first, w1, w2), jnp.where(lo_first, w2, w1)], axis=1)
    oh_ref[...] = ((iota == jnp.minimum(i1, i2))
                   | (iota == jnp.maximum(i1, i2))).astype(jnp.float32)


def _k2_body(oh_ref, dlo_ref, dhi_ref, bs_ref):
    oh = oh_ref[...]
    inc = oh
    off = 1
    while off < N:
        inc = inc + jnp.concatenate(
            [jnp.zeros((off, E), jnp.float32), inc[:-off]], axis=0)
        off *= 2
    excl = inc - oh
    counts = inc[N - 1:N, :]
    pc = jnp.ceil(counts * (1.0 / BLK)) * BLK
    ends = pc
    off = 1
    while off < E:
        ends = ends + jnp.concatenate(
            [jnp.zeros((1, off), jnp.float32), ends[:, :-off]], axis=1)
        off *= 2
    dest = (ends - pc) + excl
    active = oh > 0.0
    dlo_ref[...] = jnp.min(
        jnp.where(active, dest, 1e9), axis=1, keepdims=True).astype(jnp.int32)
    dhi_ref[...] = jnp.max(
        jnp.where(active, dest, -1.0), axis=1, keepdims=True).astype(jnp.int32)
    bs = jnp.concatenate(
        [(ends - pc) * (1.0 / BLK), ends[:, E - 1:E] * (1.0 / BLK),
         jnp.zeros((1, 128 - E - 1), jnp.float32)], axis=1)
    bs_ref[...] = bs.astype(jnp.int32)


def _sc_scatter_body(x_hbm, dlo_hbm, dhi_hbm, xg_hbm,
                     dlo_v, dhi_v, buf0, buf1, lsem0, lsem1, ssem0, ssem1):
    w = lax.axis_index("s") * 2 + lax.axis_index("c")
    base = w * TPW
    pltpu.sync_copy(dlo_hbm.at[w], dlo_v)
    pltpu.sync_copy(dhi_hbm.at[w], dhi_v)
    bufs = (buf0, buf1)
    lsems = (lsem0, lsem1)
    ssems = (ssem0, ssem1)
    loads = [None] * NSUB
    scat = []
    loads[0] = pltpu.async_copy(x_hbm.at[pl.ds(base, SUB)], bufs[0], lsems[0])
    for sub in range(NSUB):
        cur = sub & 1
        nxt = 1 - cur
        loads[sub].wait()
        if sub + 1 < NSUB:
            if sub >= 1:
                scat[2 * sub - 2].wait()
                scat[2 * sub - 1].wait()
            loads[sub + 1] = pltpu.async_copy(
                x_hbm.at[pl.ds(base + (sub + 1) * SUB, SUB)], bufs[nxt], lsems[nxt])
        scat.append(pltpu.async_copy(bufs[cur], xg_hbm.at[dlo_v.at[sub]], ssems[cur]))
        scat.append(pltpu.async_copy(bufs[cur], xg_hbm.at[dhi_v.at[sub]], ssems[cur]))
    for t in range(4):
        scat[2 * NSUB - 4 + t].wait()


def _sc_gather_body(yg_hbm, dlo_hbm, dhi_hbm, c0_hbm, c1_hbm,
                    dlo_v, dhi_v, buf0, buf1, gsem0, gsem1, wsem0, wsem1):
    w = lax.axis_index("s") * 2 + lax.axis_index("c")
    base = w * TPW
    pltpu.sync_copy(dlo_hbm.at[w], dlo_v)
    pltpu.sync_copy(dhi_hbm.at[w], dhi_v)
    bufs = (buf0, buf1)
    gsems = (gsem0, gsem1)
    wsems = (wsem0, wsem1)
    NT = 2 * NSUB

    def idx_ref(t):
        return (dlo_v if t % 2 == 0 else dhi_v).at[t // 2]

    def out_ref(t):
        dst = c0_hbm if t % 2 == 0 else c1_hbm
        return dst.at[pl.ds(base + (t // 2) * SUB, SUB)]

    g = [None] * NT
    wrt = [None] * NT
    g[0] = pltpu.async_copy(yg_hbm.at[idx_ref(0)], bufs[0], gsems[0])
    for t in range(NT):
        cur = t & 1
        nxt = 1 - cur
        g[t].wait()
        if t + 1 < NT:
            if t >= 1:
                wrt[t - 1].wait()
            g[t + 1] = pltpu.async_copy(yg_hbm.at[idx_ref(t + 1)], bufs[nxt], gsems[nxt])
        wrt[t] = pltpu.async_copy(bufs[cur], out_ref(t), wsems[cur])
    wrt[NT - 2].wait()
    wrt[NT - 1].wait()


def _sc_mesh():
    return plsc.VectorSubcoreMesh(core_axis_name="c", subcore_axis_name="s")


def _scatter_call(x, dlo3, dhi3):
    return pl.kernel(
        _sc_scatter_body,
        out_type=jax.ShapeDtypeStruct((NP, DF), jnp.float32),
        mesh=_sc_mesh(),
        scratch_types=[
            pltpu.VMEM((NSUB, SUB), jnp.int32),
            pltpu.VMEM((NSUB, SUB), jnp.int32),
            pltpu.VMEM((SUB, DF), jnp.float32),
            pltpu.VMEM((SUB, DF), jnp.float32),
            pltpu.SemaphoreType.DMA,
            pltpu.SemaphoreType.DMA,
            pltpu.SemaphoreType.DMA,
            pltpu.SemaphoreType.DMA,
        ],
    )(x, dlo3, dhi3)


def _gather_call(yg, dlo3, dhi3):
    return pl.kernel(
        _sc_gather_body,
        out_type=(
            jax.ShapeDtypeStruct((N, DF), jnp.float32),
            jax.ShapeDtypeStruct((N, DF), jnp.float32),
        ),
        mesh=_sc_mesh(),
        scratch_types=[
            pltpu.VMEM((NSUB, SUB), jnp.int32),
            pltpu.VMEM((NSUB, SUB), jnp.int32),
            pltpu.VMEM((SUB, DF), jnp.float32),
            pltpu.VMEM((SUB, DF), jnp.float32),
            pltpu.SemaphoreType.DMA,
            pltpu.SemaphoreType.DMA,
            pltpu.SemaphoreType.DMA,
            pltpu.SemaphoreType.DMA,
        ],
    )(yg, dlo3, dhi3)


def _k4_body(bs_ref, b1_ref, b2_ref, xg_ref, W1_ref, W2_ref, yg_ref,
             w1b, w2b, xb, yb, w1sem, w2sem, xsem, ysem):
    total = bs_ref[E]

    def w_copy(e, par):
        return (pltpu.make_async_copy(W1_ref.at[e], w1b.at[par], w1sem.at[par]),
                pltpu.make_async_copy(W2_ref.at[e], w2b.at[par], w2sem.at[par]))

    def x_copy(b, par):
        return pltpu.make_async_copy(
            xg_ref.at[pl.ds(b * BLK, BLK)], xb.at[par], xsem.at[par])

    def y_copy(b, par):
        return pltpu.make_async_copy(
            yb.at[par], yg_ref.at[pl.ds(b * BLK, BLK)], ysem.at[par])

    for c in w_copy(0, 0):
        c.start()
    x_copy(0, 0).start()

    for e in range(E):
        par = e & 1
        for c in w_copy(e, par):
            c.wait()
        if e + 1 < E:
            for c in w_copy(e + 1, 1 - par):
                c.start()

        def body(b, carry, par=par, e=e):
            bp = lax.rem(b, 2)
            x_copy(b, bp).wait()

            @pl.when(b + 1 < total)
            def _():
                x_copy(b + 1, 1 - bp).start()

            @pl.when(b >= 2)
            def _():
                y_copy(b - 2, bp).wait()

            x = xb[bp]
            u = jnp.dot(x, w1b[par], preferred_element_type=jnp.float32) + b1_ref[e]
            h = 0.5 * u * (1.0 + jnp.tanh(
                0.7978845608028654 * (u + 0.044715 * u * u * u)))
            yb[bp] = (jnp.dot(h, w2b[par], preferred_element_type=jnp.float32)
                      + b2_ref[e])
            y_copy(b, bp).start()
            return carry

        lax.fori_loop(bs_ref[e], bs_ref[e + 1], body, 0)

    y_copy(0, lax.rem(total, 2)).wait()
    y_copy(0, lax.rem(total + 1, 2)).wait()


def _k6_body(c0_ref, c1_ref, wp_ref, mk_ref, Wo_ref, bo_ref, y_ref):
    wp = wp_ref[...]
    comb = (wp[:, :1] * c0_ref[...] + wp[:, 1:2] * c1_ref[...]) * mk_ref[...]
    y_ref[...] = (jnp.dot(comb, Wo_ref[...], preferred_element_type=jnp.float32)
                  + bo_ref[...])


def kernel(h2d, h3d, mask, W_in, b_in, W_gate, b_gate, W1, b1, W2, b2, W_out, b_out):
    h2 = h2d.reshape(N, D2D)
    h3 = h3d.reshape(N, D3D)
    maskf = mask.reshape(N, 1).astype(jnp.float32)

    x, oh, wp = pl.pallas_call(
        _k1_body,
        grid=(N // BT,),
        in_specs=[
            pl.BlockSpec((BT, D2D), lambda i: (i, 0)),
            pl.BlockSpec((BT, D3D), lambda i: (i, 0)),
            pl.BlockSpec((D2D + D3D, DF), lambda i: (0, 0)),
            pl.BlockSpec((1, DF), lambda i: (0, 0)),
            pl.BlockSpec((DF, E), lambda i: (0, 0)),
            pl.BlockSpec((1, E), lambda i: (0, 0)),
        ],
        out_specs=[
            pl.BlockSpec((BT, DF), lambda i: (i, 0)),
            pl.BlockSpec((BT, E), lambda i: (i, 0)),
            pl.BlockSpec((BT, 2), lambda i: (i, 0)),
        ],
        out_shape=[
            jax.ShapeDtypeStruct((N, DF), jnp.float32),
            jax.ShapeDtypeStruct((N, E), jnp.float32),
            jax.ShapeDtypeStruct((N, 2), jnp.float32),
        ],
    )(h2, h3, W_in, b_in.reshape(1, DF), W_gate, b_gate.reshape(1, E))

    dlo, dhi, bs2 = pl.pallas_call(
        _k2_body,
        out_shape=[
            jax.ShapeDtypeStruct((N, 1), jnp.int32),
            jax.ShapeDtypeStruct((N, 1), jnp.int32),
            jax.ShapeDtypeStruct((1, 128), jnp.int32),
        ],
    )(oh)

    bs = bs2.reshape(128)[:16]
    dlo3 = dlo.reshape(NW, NSUB, SUB)
    dhi3 = dhi.reshape(NW, NSUB, SUB)

    xg = _scatter_call(x, dlo3, dhi3)

    yg = pl.pallas_call(
        _k4_body,
        in_specs=[
            pl.BlockSpec(memory_space=pltpu.SMEM),
            pl.BlockSpec((E, DFF), lambda: (0, 0)),
            pl.BlockSpec((E, DF), lambda: (0, 0)),
            pl.BlockSpec(memory_space=pl.ANY),
            pl.BlockSpec(memory_space=pl.ANY),
            pl.BlockSpec(memory_space=pl.ANY),
        ],
        out_specs=pl.BlockSpec(memory_space=pl.ANY),
        out_shape=jax.ShapeDtypeStruct((NP, DF), jnp.float32),
        scratch_shapes=[
            pltpu.VMEM((2, DF, DFF), jnp.float32),
            pltpu.VMEM((2, DFF, DF), jnp.float32),
            pltpu.VMEM((2, BLK, DF), jnp.float32),
            pltpu.VMEM((2, BLK, DF), jnp.float32),
            pltpu.SemaphoreType.DMA((2,)),
            pltpu.SemaphoreType.DMA((2,)),
            pltpu.SemaphoreType.DMA((2,)),
            pltpu.SemaphoreType.DMA((2,)),
        ],
    )(bs, b1, b2, xg, W1, W2)

    c0, c1 = _gather_call(yg, dlo3, dhi3)

    y = pl.pallas_call(
        _k6_body,
        grid=(N // BT,),
        in_specs=[
            pl.BlockSpec((BT, DF), lambda i: (i, 0)),
            pl.BlockSpec((BT, DF), lambda i: (i, 0)),
            pl.BlockSpec((BT, 2), lambda i: (i, 0)),
            pl.BlockSpec((BT, 1), lambda i: (i, 0)),
            pl.BlockSpec((DF, OUT), lambda i: (0, 0)),
            pl.BlockSpec((1, OUT), lambda i: (0, 0)),
        ],
        out_specs=pl.BlockSpec((BT, OUT), lambda i: (i, 0)),
        out_shape=jax.ShapeDtypeStruct((N, OUT), jnp.float32),
    )(c0, c1, wp, maskf, W_out, b_out.reshape(1, OUT))

    return y.reshape(B, A, OUT)
